# Initial kernel scaffold; baseline (speedup 1.0000x reference)
import jax
import jax.numpy as jnp
from jax.experimental import pallas as pl


def kernel(actions, emb_weight):
    raise NotImplementedError("write your pallas kernel here")


# speedup vs baseline: 5.5600x; 5.5600x over previous
import functools

import jax
import jax.numpy as jnp
from jax import lax
from jax.experimental import pallas as pl
from jax.experimental.pallas import tpu as pltpu
from jax.experimental.pallas import tpu_sc as plsc

_BATCH = 4096
_SEQ = 200
_D = 128
_B = _BATCH * _SEQ
_NW = 32
_B_PER_W = _B // _NW
_CHUNK = 128
_N_CHUNKS = _B_PER_W // _CHUNK


def _emb_body(idx_hbm, table_hbm, out_hbm, idx_v, rows_v, gsem):
    wid = lax.axis_index("s") * 2 + lax.axis_index("c")
    base = wid * _B_PER_W
    pltpu.sync_copy(idx_hbm.at[wid], idx_v)

    def body(j, _):
        pltpu.async_copy(table_hbm.at[idx_v.at[j]], rows_v, gsem).wait()
        pltpu.sync_copy(rows_v, out_hbm.at[pl.ds(base + j * _CHUNK, _CHUNK)])
        return 0

    lax.fori_loop(0, _N_CHUNKS, body, 0)


_emb_kernel = functools.partial(
    pl.kernel,
    out_type=jax.ShapeDtypeStruct((_B, _D), jnp.float32),
    mesh=plsc.VectorSubcoreMesh(core_axis_name="c", subcore_axis_name="s"),
    scratch_types=[
        pltpu.VMEM((_N_CHUNKS, _CHUNK), jnp.int32),
        pltpu.VMEM((_CHUNK, _D), jnp.float32),
        pltpu.SemaphoreType.DMA,
    ],
)(_emb_body)


def kernel(actions, emb_weight):
    idx = actions.reshape(_NW, _N_CHUNKS, _CHUNK).astype(jnp.int32)
    out = _emb_kernel(idx, emb_weight)
    return out.reshape(_BATCH, _SEQ, _D)


# speedup vs baseline: 6.7605x; 1.2159x over previous
import functools

import jax
import jax.numpy as jnp
from jax import lax
from jax.experimental import pallas as pl
from jax.experimental.pallas import tpu as pltpu
from jax.experimental.pallas import tpu_sc as plsc

_BATCH = 4096
_SEQ = 200
_D = 128
_B = _BATCH * _SEQ
_NW = 32
_B_PER_W = _B // _NW
_CHUNK = 128
_N_CHUNKS = _B_PER_W // _CHUNK
_NBUF = 4


def _emb_body(idx_hbm, table_hbm, out_hbm, idx_v, rows, gsems, ssems):
    wid = lax.axis_index("s") * 2 + lax.axis_index("c")
    base = wid * _B_PER_W
    pltpu.sync_copy(idx_hbm.at[wid], idx_v)

    def gather(j, b):
        pltpu.async_copy(table_hbm.at[idx_v.at[j]], rows[b], gsems[b])

    def gwait(b):
        pltpu.make_async_copy(table_hbm.at[idx_v.at[0]], rows[b],
                              gsems[b]).wait()

    def store(j, b):
        pltpu.async_copy(rows[b],
                         out_hbm.at[pl.ds(base + j * _CHUNK, _CHUNK)],
                         ssems[b])

    def swait(b):
        pltpu.make_async_copy(rows[b],
                              out_hbm.at[pl.ds(base, _CHUNK)],
                              ssems[b]).wait()

    for j in range(3):
        gather(j, j)
    gather(3, 3)
    gwait(0)
    store(0, 0)
    for j in (1, 2, 3):
        swait(j - 1)
        gather(j + 3, (j + 3) % _NBUF)
        gwait(j)
        store(j, j)

    def body(t, _):
        for b in range(_NBUF):
            j = t * _NBUF + b
            swait((b + 3) % _NBUF)
            gather(j + 3, (b + 3) % _NBUF)
            gwait(b)
            store(j, b)
        return 0

    lax.fori_loop(1, _N_CHUNKS // _NBUF - 1, body, 0)

    swait(3)
    gather(199, 3)
    gwait(0)
    store(196, 0)
    for j, b in ((197, 1), (198, 2), (199, 3)):
        gwait(b)
        store(j, b)
    for b in range(_NBUF):
        swait(b)


_emb_kernel = functools.partial(
    pl.kernel,
    out_type=jax.ShapeDtypeStruct((_B, _D), jnp.float32),
    mesh=plsc.VectorSubcoreMesh(core_axis_name="c", subcore_axis_name="s"),
    scratch_types=[
        pltpu.VMEM((_N_CHUNKS, _CHUNK), jnp.int32),
        [pltpu.VMEM((_CHUNK, _D), jnp.float32)] * _NBUF,
        [pltpu.SemaphoreType.DMA] * _NBUF,
        [pltpu.SemaphoreType.DMA] * _NBUF,
    ],
)(_emb_body)


def kernel(actions, emb_weight):
    idx = actions.reshape(_NW, _N_CHUNKS, _CHUNK).astype(jnp.int32)
    out = _emb_kernel(idx, emb_weight)
    return out.reshape(_BATCH, _SEQ, _D)
